# baseline (speedup 1.0000x reference)
import functools

import jax
import jax.numpy as jnp
from jax import lax
from jax.experimental import pallas as pl
from jax.experimental.pallas import tpu as pltpu
from jax.experimental.pallas import tpu_sc as plsc

ROWS = 16384
WALK = 50
EMB = 64
IN_DIM = 1000000
B = ROWS * WALK
NC, NS = 2, 16
NW = NC * NS

W_A = 512
P_A = W_A // 2
NBLK_A = IN_DIM // W_A
BASE_BLKS = NBLK_A // NW
EXTRA_BLKS = NBLK_A % NW
TAIL_N0 = NBLK_A * W_A
TAIL_W = IN_DIM - TAIL_N0
TAIL_P = TAIL_W // 2

CHUNK = 256
NCHUNK = (B // NW) // CHUNK

_mesh = plsc.VectorSubcoreMesh(core_axis_name="c", subcore_axis_name="s")


@functools.partial(
    pl.kernel,
    mesh=_mesh,
    compiler_params=pltpu.CompilerParams(needs_layout_passes=False),
    out_type=jax.ShapeDtypeStruct((IN_DIM // 2, 2 * EMB), jnp.float32),
    scratch_types=[
        pltpu.VMEM((EMB, W_A), jnp.float32),
        pltpu.VMEM((P_A, 2 * EMB), jnp.float32),
        pltpu.VMEM((EMB, TAIL_W), jnp.float32),
    ],
)
def _relayout(wt_hbm, tail_hbm, fused_hbm, in_blk, out_blk, in_tail):
    wid = lax.axis_index("s") * NC + lax.axis_index("c")
    nblk = BASE_BLKS + jnp.where(wid < EXTRA_BLKS, 1, 0)
    row_qs = [lax.iota(jnp.int32, 16) + 16 * v for v in range(8)]

    def do_block(src, p0, n_p):
        def tpose(p, _):
            for v in range(8):
                h = v // 4
                rows = row_qs[v] - h * EMB
                col = jnp.full((16,), 2 * p + h, jnp.int32)
                vals = plsc.load_gather(src, [rows, col])
                out_blk[p, pl.ds(16 * v, 16)] = vals
            return 0

        lax.fori_loop(0, n_p, tpose, 0)
        pltpu.sync_copy(
            out_blk.at[pl.ds(0, n_p)], fused_hbm.at[pl.ds(p0, n_p)]
        )

    def body(i, _):
        blk = wid + NW * i
        n0 = blk * W_A
        pltpu.sync_copy(wt_hbm.at[:, pl.ds(n0, W_A)], in_blk)
        do_block(in_blk, blk * P_A, P_A)
        return 0

    lax.fori_loop(0, nblk, body, 0)

    @pl.when(wid == 1)
    def _tail():
        pltpu.sync_copy(tail_hbm, in_tail)
        do_block(in_tail, TAIL_N0 // 2, TAIL_P)


@functools.partial(
    pl.kernel,
    mesh=_mesh,
    compiler_params=pltpu.CompilerParams(needs_layout_passes=False),
    out_type=jax.ShapeDtypeStruct((WALK, EMB, ROWS), jnp.float32),
    scratch_types=[
        pltpu.VMEM((CHUNK,), jnp.int32),
        pltpu.VMEM((CHUNK,), jnp.int32),
        pltpu.VMEM((CHUNK, 2 * EMB), jnp.float32),
        pltpu.VMEM((EMB, CHUNK), jnp.float32),
        pltpu.SemaphoreType.DMA,
    ],
)
def _gather(fused_hbm, xt_hbm, out_hbm, idx_v, j_v, rows2_v, out_blk, sem):
    wid = lax.axis_index("s") * NC + lax.axis_index("c")
    qbase = wid * (B // NW)
    lanes = lax.iota(jnp.int32, 16)

    def body(i, _):
        q0 = qbase + i * CHUNK
        w = q0 // ROWS
        r0 = q0 % ROWS
        pltpu.sync_copy(xt_hbm.at[w, pl.ds(r0, CHUNK)], idx_v)

        def shift(g, _):
            s = pl.ds(g * 16, 16)
            j_v[s] = lax.shift_right_logical(idx_v[s], 1)
            return 0

        lax.fori_loop(0, CHUNK // 16, shift, 0)
        pltpu.async_copy(fused_hbm.at[j_v], rows2_v, sem).wait()

        def select(g, _):
            s = pl.ds(g * 16, 16)
            row_v = g * 16 + lanes
            h64 = (idx_v[s] & 1) * EMB
            for f in range(EMB):
                vals = plsc.load_gather(rows2_v, [row_v, h64 + f])
                out_blk[f, pl.ds(g * 16, 16)] = vals
            return 0

        lax.fori_loop(0, CHUNK // 16, select, 0)
        pltpu.sync_copy(out_blk, out_hbm.at[w, :, pl.ds(r0, CHUNK)])
        return 0

    lax.fori_loop(0, NCHUNK, body, 0)


def kernel(x, node_embeddings):
    wt = node_embeddings.T
    xt = x.astype(jnp.int32).T
    tail = wt[:, TAIL_N0:]
    fused = _relayout(wt, tail)
    out3 = _gather(fused, xt)
    return jnp.transpose(out3, (2, 0, 1))
